# baseline (speedup 1.0000x reference)
import functools

import jax
import jax.numpy as jnp
from jax import lax
from jax.experimental import pallas as pl
from jax.experimental.pallas import tpu as pltpu
from jax.experimental.pallas import tpu_sc as plsc

S, B, W = 16, 128, 16384
L = 16
NC, NS = 2, 16
NW = NC * NS
B_PER_W = B // NW
C = 2048
NCHUNK = W // C
U = 4

RTOL = 1e-5
ATOL = 1e-8


def _make_sc_call():
    mesh = plsc.VectorSubcoreMesh(core_axis_name="c", subcore_axis_name="s")

    scratch = [
        pltpu.VMEM((L,), jnp.float32),
        pltpu.VMEM((C,), jnp.int32),
        pltpu.VMEM((C,), jnp.int32),
        pltpu.VMEM((C,), jnp.float32),
        pltpu.VMEM((C,), jnp.float32),
        pltpu.VMEM((S, C), jnp.float32),
        pltpu.VMEM((S, C), jnp.float32),
        pltpu.VMEM((B_PER_W, L), jnp.int32),
        pltpu.SemaphoreType.DMA,
        pltpu.SemaphoreType.DMA,
    ]

    @functools.partial(
        pl.kernel,
        out_type=jax.ShapeDtypeStruct((B, S), jnp.int32),
        mesh=mesh,
        scratch_types=scratch,
    )
    def body(y_sby_hbm, y_by_hbm, w_hbm, costs_hbm, out_hbm,
             costs_v, w0, w1, yb0, yb1, ys0, ys1, res, sem0, sem1):
        wid = lax.axis_index("s") * NC + lax.axis_index("c")
        b_lo = wid * B_PER_W
        bufs = [(w0, yb0, ys0, sem0), (w1, yb1, ys1, sem1)]

        pltpu.sync_copy(costs_hbm, costs_v)
        cost_tbl = costs_v[...]

        def gatherv(v, idx):
            return lax.gather(
                v, idx[:, None],
                dimension_numbers=lax.GatherDimensionNumbers(
                    offset_dims=(), collapsed_slice_dims=(0,),
                    start_index_map=(0,)),
                slice_sizes=(1,),
                mode=lax.GatherScatterMode.PROMISE_IN_BOUNDS)

        lane = lax.iota(jnp.int32, L)
        masks = {d: (lane & d) == 0 for d in (1, 2, 4, 8)}
        xors = {d: lane ^ d for d in (1, 2, 4, 8)}

        def combine(x, y, d):
            p = jnp.where(masks[d], x, y)
            q = gatherv(jnp.where(masks[d], y, x), xors[d])
            return p + q

        def hsum_splat(v):
            for d in (1, 2, 4, 8):
                v = v + gatherv(v, xors[d])
            return v

        def start_chunk(slot, b, c):
            w_ref, yb_ref, ys_ref, sem = slot
            off = c * C
            pltpu.async_copy(w_hbm.at[b, pl.ds(off, C)], w_ref, sem)
            pltpu.async_copy(y_by_hbm.at[b, pl.ds(off, C)], yb_ref, sem)
            pltpu.async_copy(y_sby_hbm.at[:, b, pl.ds(off, C)], ys_ref, sem)

        def wait_chunk(slot):
            w_ref, yb_ref, ys_ref, sem = slot
            pltpu.make_async_copy(w_hbm.at[0, pl.ds(0, C)], w_ref, sem).wait()
            pltpu.make_async_copy(y_by_hbm.at[0, pl.ds(0, C)], yb_ref, sem).wait()
            pltpu.make_async_copy(y_sby_hbm.at[:, 0, pl.ds(0, C)], ys_ref,
                                  sem).wait()

        def accum_chunk(slot, accs):
            w_ref, yb_ref, ys_ref, _ = slot

            def step(jj, carry):
                gt = carry[0]
                hats = list(carry[1:])
                for u in range(U):
                    base = (jj * U + u) * L
                    wv = w_ref[pl.ds(base, L)]
                    cost = gatherv(cost_tbl, wv)
                    gt = gt + cost * yb_ref[pl.ds(base, L)]
                    for s in range(S):
                        hats[s] = hats[s] + cost * ys_ref[s, pl.ds(base, L)]
                return (gt, *hats)

            return lax.fori_loop(0, C // (L * U), step, accs)

        def finish_batch(bi, accs):
            gt_sum = hsum_splat(accs[0])
            vs = list(accs[1:])
            for d in (1, 2, 4, 8):
                vs = [combine(vs[2 * i], vs[2 * i + 1], d)
                      for i in range(len(vs) // 2)]
            hat = vs[0]
            ok = jnp.abs(gt_sum - hat) <= ATOL + RTOL * jnp.abs(hat)
            res[bi, :] = jnp.where(ok, 1, 0).astype(jnp.int32)

        zeros = (jnp.zeros((L,), jnp.float32),) * (S + 1)

        start_chunk(bufs[0], b_lo, 0)

        for bi in range(B_PER_W):
            b = b_lo + bi
            last_batch = bi == B_PER_W - 1

            def pair_body(p, accs, b=b, last_batch=last_batch):
                start_chunk(bufs[1], b, 2 * p + 1)
                wait_chunk(bufs[0])
                accs = accum_chunk(bufs[0], accs)
                nxt = 2 * p + 2
                if last_batch:
                    @pl.when(p < NCHUNK // 2 - 1)
                    def _():
                        start_chunk(bufs[0], b, nxt)
                else:
                    start_chunk(bufs[0], b + nxt // NCHUNK, nxt % NCHUNK)
                wait_chunk(bufs[1])
                return accum_chunk(bufs[1], accs)

            accs = lax.fori_loop(0, NCHUNK // 2, pair_body, zeros)
            finish_batch(bi, accs)

        pltpu.sync_copy(res, out_hbm.at[pl.ds(b_lo, B_PER_W), :])

    return body


_sc_call = _make_sc_call()


@jax.jit
def kernel(y_0_SBY, y_0_BY, w_0_BW, costs_t):
    costs16 = jnp.concatenate(
        [costs_t.astype(jnp.float32), jnp.zeros((L - 4,), jnp.float32)])
    out_bs = _sc_call(y_0_SBY, y_0_BY, w_0_BW, costs16)
    return out_bs.T.astype(bool)
